# baseline (speedup 1.0000x reference)
import functools

import jax
import jax.numpy as jnp
from jax import lax
from jax.experimental import pallas as pl
from jax.experimental.layout import Layout, with_layout_constraint
from jax.experimental.pallas import tpu as pltpu
from jax.experimental.pallas import tpu_sc as plsc

VOCAB = 1_000_000
D = 64
DP = 128

_info = plsc.get_sparse_core_info()
NC = _info.num_cores
NS = _info.num_subcores
NW = NC * NS

VBLK = 128
NFULL = VOCAB // VBLK
VMAIN = NFULL * VBLK
BPW = 246

CHUNK = 64
NBUF = 8
K = 4


def _transpose_kernel():
    mesh = plsc.VectorSubcoreMesh(core_axis_name="c", subcore_axis_name="s")

    @functools.partial(
        pl.kernel,
        out_type=jax.ShapeDtypeStruct((VOCAB, DP), jnp.float32),
        mesh=mesh,
        scratch_types=[
            pltpu.VMEM((2, D, VBLK), jnp.float32),
            pltpu.VMEM((2, VBLK, DP), jnp.float32),
        ]
        + [pltpu.SemaphoreType.DMA] * 4,
        compiler_params=pltpu.CompilerParams(needs_layout_passes=False),
    )
    def tr(wt_hbm, tail_hbm, tab_hbm, in_v, out_v, isem0, isem1, osem0, osem1):
        isems = (isem0, isem1)
        osems = (osem0, osem1)
        wid = lax.axis_index("s") * NC + lax.axis_index("c")

        def blk(t):
            return t * NW + wid

        def live(t):
            return (blk(t) >= 0) & (blk(t) < NFULL)

        def start_in(t, s):
            @pl.when(live(t))
            def _():
                pltpu.async_copy(
                    wt_hbm.at[:, pl.ds(blk(t) * VBLK, VBLK)], in_v.at[s], isems[s]
                )

        def wait_in(t, s):
            @pl.when(live(t))
            def _():
                pltpu.make_async_copy(
                    wt_hbm.at[:, pl.ds(0, VBLK)], in_v.at[s], isems[s]
                ).wait()

        def start_out(t, s):
            @pl.when(live(t))
            def _():
                pltpu.async_copy(
                    out_v.at[s],
                    tab_hbm.at[pl.ds(blk(t) * VBLK, VBLK)],
                    osems[s],
                )

        def wait_out(t, s):
            @pl.when(live(t))
            def _():
                pltpu.make_async_copy(
                    out_v.at[s],
                    tab_hbm.at[pl.ds(0, VBLK)],
                    osems[s],
                ).wait()

        def compute(t, s):
            @pl.when(live(t))
            def _():
                def row(vv, _):
                    cols = jnp.full((16,), vv, dtype=jnp.int32)
                    for dg in range(D // 16):
                        rows = dg * 16 + lax.iota(jnp.int32, 16)
                        vals = plsc.load_gather(in_v.at[s], [rows, cols])
                        out_v[s, vv, pl.ds(dg * 16, 16)] = vals
                    return 0

                lax.fori_loop(0, VBLK, row, 0)

        start_in(0, 0)
        start_in(1, 1)

        def body(t2, _):
            for b in range(2):
                t = 2 * t2 + b
                wait_in(t, b)
                wait_out(t - 2, b)
                compute(t, b)
                start_out(t, b)
                start_in(t + 2, b)
            return 0

        lax.fori_loop(0, BPW // 2, body, 0)
        for b in range(2):
            wait_out(BPW - 2 + b, b)

        @pl.when(wid == 0)
        def _():
            pltpu.sync_copy(tail_hbm, in_v.at[0])
            pltpu.sync_copy(in_v.at[0], tab_hbm.at[pl.ds(VMAIN, VOCAB - VMAIN)])

    return tr


def _gather_kernel(B):
    assert B % NW == 0
    b_per_w = B // NW
    assert b_per_w % CHUNK == 0
    ncnk = b_per_w // CHUNK
    T = ncnk // NBUF
    assert ncnk % NBUF == 0 and T >= 2

    mesh = plsc.VectorSubcoreMesh(core_axis_name="c", subcore_axis_name="s")

    @functools.partial(
        pl.kernel,
        out_type=jax.ShapeDtypeStruct((B, DP), jnp.float32),
        mesh=mesh,
        scratch_types=[
            pltpu.VMEM((ncnk, CHUNK), jnp.int32),
            pltpu.VMEM((NBUF, CHUNK, DP), jnp.float32),
        ]
        + [pltpu.SemaphoreType.DMA] * (2 * NBUF),
        compiler_params=pltpu.CompilerParams(use_tc_tiling_on_sc=False),
    )
    def emb(idx_hbm, table_hbm, out_hbm, idx_v, rows_v, *sems):
        gsems = sems[:NBUF]
        osems = sems[NBUF:]
        wid = lax.axis_index("s") * NC + lax.axis_index("c")
        base = wid * b_per_w

        pltpu.sync_copy(idx_hbm.at[wid], idx_v)

        def start_gather(g, s):
            pltpu.async_copy(table_hbm.at[idx_v.at[g]], rows_v.at[s], gsems[s])

        def wait_gather(s):
            pltpu.make_async_copy(
                table_hbm.at[idx_v.at[0]], rows_v.at[s], gsems[s]
            ).wait()

        def start_out(g, s):
            pltpu.async_copy(
                rows_v.at[s],
                out_hbm.at[pl.ds(base + g * CHUNK, CHUNK)],
                osems[s],
            )

        def wait_out(s):
            pltpu.make_async_copy(
                rows_v.at[s],
                out_hbm.at[pl.ds(0, CHUNK)],
                osems[s],
            ).wait()

        def visit(g, b, first=False, last=False):
            s2 = (b + K) % NBUF
            if (not first) or (b >= NBUF - K):
                wait_out(s2)
            if (not last) or (b < NBUF - K):
                start_gather(g + K, s2)
            wait_gather(b)
            start_out(g, b)

        for c in range(K):
            start_gather(c, c)

        for b in range(NBUF):
            visit(b, b, first=True)

        def outer(t, _):
            for b in range(NBUF):
                visit(t * NBUF + b, b)
            return 0

        lax.fori_loop(1, T - 1, outer, 0)

        for b in range(NBUF):
            visit(ncnk - NBUF + b, b, last=True)
        for g in range(ncnk - (NBUF - K), ncnk):
            wait_out(g % NBUF)

    return emb


@jax.jit
def kernel(token_ids, weight):
    shape = token_ids.shape
    B = 1
    for s in shape:
        B *= s
    idx = token_ids.reshape(NW, B // (NW * CHUNK), CHUNK).astype(jnp.int32)
    tail = jnp.pad(weight[VMAIN:], ((0, 0), (0, DP - D)))
    table = _transpose_kernel()(weight.T, tail)
    out = _gather_kernel(B)(idx, table)
    out = out[:, :D].reshape(*shape, D)
    return with_layout_constraint(out, Layout((2, 1, 0)))


# speedup vs baseline: 1.1594x; 1.1594x over previous
import functools

import jax
import jax.numpy as jnp
from jax import lax
from jax.experimental import pallas as pl
from jax.experimental.layout import Layout, with_layout_constraint
from jax.experimental.pallas import tpu as pltpu
from jax.experimental.pallas import tpu_sc as plsc

VOCAB = 1_000_000
D = 64
DP = 128

_info = plsc.get_sparse_core_info()
NC = _info.num_cores
NS = _info.num_subcores
NW = NC * NS

VBLK = 128
NFULL = VOCAB // VBLK
VMAIN = NFULL * VBLK
BPW = 246

CHUNK = 64
NBUF = 8
K = 4


def _transpose_kernel():
    mesh = plsc.VectorSubcoreMesh(core_axis_name="c", subcore_axis_name="s")

    @functools.partial(
        pl.kernel,
        out_type=jax.ShapeDtypeStruct((VOCAB, DP), jnp.float32),
        mesh=mesh,
        scratch_types=[
            pltpu.VMEM((2, D, VBLK), jnp.float32),
            pltpu.VMEM((2, VBLK, DP), jnp.float32),
        ]
        + [pltpu.SemaphoreType.DMA] * 4,
        compiler_params=pltpu.CompilerParams(needs_layout_passes=False),
    )
    def tr(wt_hbm, tail_hbm, tab_hbm, in_v, out_v, isem0, isem1, osem0, osem1):
        isems = (isem0, isem1)
        osems = (osem0, osem1)
        wid = lax.axis_index("s") * NC + lax.axis_index("c")

        def blk(t):
            return t * NW + wid

        def live(t):
            return (blk(t) >= 0) & (blk(t) < NFULL)

        def start_in(t, s):
            @pl.when(live(t))
            def _():
                pltpu.async_copy(
                    wt_hbm.at[:, pl.ds(blk(t) * VBLK, VBLK)], in_v.at[s], isems[s]
                )

        def wait_in(t, s):
            @pl.when(live(t))
            def _():
                pltpu.make_async_copy(
                    wt_hbm.at[:, pl.ds(0, VBLK)], in_v.at[s], isems[s]
                ).wait()

        def start_out(t, s):
            @pl.when(live(t))
            def _():
                pltpu.async_copy(
                    out_v.at[s],
                    tab_hbm.at[pl.ds(blk(t) * VBLK, VBLK)],
                    osems[s],
                )

        def wait_out(t, s):
            @pl.when(live(t))
            def _():
                pltpu.make_async_copy(
                    out_v.at[s],
                    tab_hbm.at[pl.ds(0, VBLK)],
                    osems[s],
                ).wait()

        rows_c = [vg * 16 + lax.iota(jnp.int32, 16) for vg in range(VBLK // 16)]

        def compute(t, s):
            @pl.when(live(t))
            def _():
                for d in range(D):
                    cols = jnp.full((16,), d, dtype=jnp.int32)
                    for vg in range(VBLK // 16):
                        vals = in_v[s, d, pl.ds(vg * 16, 16)]
                        plsc.store_scatter(out_v.at[s], [rows_c[vg], cols], vals)

        start_in(0, 0)
        start_in(1, 1)

        def body(t2, _):
            for b in range(2):
                t = 2 * t2 + b
                wait_in(t, b)
                wait_out(t - 2, b)
                compute(t, b)
                start_out(t, b)
                start_in(t + 2, b)
            return 0

        lax.fori_loop(0, BPW // 2, body, 0)
        for b in range(2):
            wait_out(BPW - 2 + b, b)

        @pl.when(wid == 0)
        def _():
            pltpu.sync_copy(tail_hbm, in_v.at[0])
            pltpu.sync_copy(in_v.at[0], tab_hbm.at[pl.ds(VMAIN, VOCAB - VMAIN)])

    return tr


def _gather_kernel(B):
    assert B % NW == 0
    b_per_w = B // NW
    assert b_per_w % CHUNK == 0
    ncnk = b_per_w // CHUNK
    T = ncnk // NBUF
    assert ncnk % NBUF == 0 and T >= 2

    mesh = plsc.VectorSubcoreMesh(core_axis_name="c", subcore_axis_name="s")

    @functools.partial(
        pl.kernel,
        out_type=jax.ShapeDtypeStruct((B, DP), jnp.float32),
        mesh=mesh,
        scratch_types=[
            pltpu.VMEM((ncnk, CHUNK), jnp.int32),
            pltpu.VMEM((NBUF, CHUNK, DP), jnp.float32),
        ]
        + [pltpu.SemaphoreType.DMA] * (2 * NBUF),
        compiler_params=pltpu.CompilerParams(use_tc_tiling_on_sc=False),
    )
    def emb(idx_hbm, table_hbm, out_hbm, idx_v, rows_v, *sems):
        gsems = sems[:NBUF]
        osems = sems[NBUF:]
        wid = lax.axis_index("s") * NC + lax.axis_index("c")
        base = wid * b_per_w

        pltpu.sync_copy(idx_hbm.at[wid], idx_v)

        def start_gather(g, s):
            pltpu.async_copy(table_hbm.at[idx_v.at[g]], rows_v.at[s], gsems[s])

        def wait_gather(s):
            pltpu.make_async_copy(
                table_hbm.at[idx_v.at[0]], rows_v.at[s], gsems[s]
            ).wait()

        def start_out(g, s):
            pltpu.async_copy(
                rows_v.at[s],
                out_hbm.at[pl.ds(base + g * CHUNK, CHUNK)],
                osems[s],
            )

        def wait_out(s):
            pltpu.make_async_copy(
                rows_v.at[s],
                out_hbm.at[pl.ds(0, CHUNK)],
                osems[s],
            ).wait()

        def visit(g, b, first=False, last=False):
            s2 = (b + K) % NBUF
            if (not first) or (b >= NBUF - K):
                wait_out(s2)
            if (not last) or (b < NBUF - K):
                start_gather(g + K, s2)
            wait_gather(b)
            start_out(g, b)

        for c in range(K):
            start_gather(c, c)

        for b in range(NBUF):
            visit(b, b, first=True)

        def outer(t, _):
            for b in range(NBUF):
                visit(t * NBUF + b, b)
            return 0

        lax.fori_loop(1, T - 1, outer, 0)

        for b in range(NBUF):
            visit(ncnk - NBUF + b, b, last=True)
        for g in range(ncnk - (NBUF - K), ncnk):
            wait_out(g % NBUF)

    return emb


@jax.jit
def kernel(token_ids, weight):
    shape = token_ids.shape
    B = 1
    for s in shape:
        B *= s
    idx = token_ids.reshape(NW, B // (NW * CHUNK), CHUNK).astype(jnp.int32)
    tail = jnp.pad(weight[VMAIN:], ((0, 0), (0, DP - D)))
    table = _transpose_kernel()(weight.T, tail)
    out = _gather_kernel(B)(idx, table)
    out = out[:, :D].reshape(*shape, D)
    return with_layout_constraint(out, Layout((2, 1, 0)))
